# baseline (speedup 1.0000x reference)
import jax
import jax.numpy as jnp
from jax.experimental import pallas as pl
from jax.experimental.pallas import tpu as pltpu


def _gate_kernel(T, E, C, NB, Tb):
    def body(x_ref, wt_ref, comb_ref, disp_ref, logits_sc, p_sc, probs_sc):
        i = pl.program_id(0)

        @pl.when(i < NB)
        def _matmul_phase():
            xb = x_ref[...]
            lg = jax.lax.dot_general(
                xb, wt_ref[...], (((1,), (0,)), ((), ())),
                preferred_element_type=jnp.float32)
            logits_sc[pl.ds(i * Tb, Tb), :] = lg

        @pl.when(i == NB - 1)
        def _gating_phase():
            logits = logits_sc[...]
            mx = jnp.max(logits, axis=1, keepdims=True)
            ex = jnp.exp(logits - mx)
            den = jnp.sum(ex, axis=1, keepdims=True)
            gates = ex / den
            idx = jax.lax.broadcasted_iota(jnp.int32, (T, E), 1)
            m1 = jnp.max(gates, axis=1, keepdims=True)
            t1 = jnp.min(jnp.where(gates == m1, idx, E), axis=1, keepdims=True)
            em1 = idx == t1
            g2 = jnp.where(em1, -1.0, gates)
            m2 = jnp.max(g2, axis=1, keepdims=True)
            t2 = jnp.min(jnp.where(g2 == m2, idx, E), axis=1, keepdims=True)
            em2 = idx == t2
            gs = jnp.maximum(m1 + m2, jnp.finfo(jnp.float32).eps)
            probs_sc[...] = gates / gs
            cnt = jnp.concatenate(
                [em1.astype(jnp.int32), em2.astype(jnp.int32)], axis=1)
            c = cnt
            s = 1
            while s < T:
                c = c + jnp.concatenate(
                    [jnp.zeros((s, 2 * E), jnp.int32), c[:T - s, :]], axis=0)
                s *= 2
            inc1 = c[:, :E]
            inc2 = c[:, E:]
            excl1 = inc1 - cnt[:, :E]
            excl2 = inc2 - cnt[:, E:]
            total1 = inc1[T - 1:T, :]
            p = jnp.where(em1, excl1,
                          jnp.where(em2, total1 + excl2, -1))
            p_sc[...] = p

        @pl.when(i >= NB)
        def _write_phase():
            bb = i - NB
            pb = p_sc[pl.ds(bb * Tb, Tb), :]
            prb = probs_sc[pl.ds(bb * Tb, Tb), :]
            vb = jnp.logical_and(pb >= 0, pb < C)
            pc = jnp.where(vb, pb, 0)
            ci = jax.lax.broadcasted_iota(jnp.int32, (Tb, C), 1)
            pieces_m = []
            pieces_c = []
            for e in range(E):
                me = jnp.logical_and(ci == pc[:, e:e + 1], vb[:, e:e + 1])
                pieces_m.append(me)
                pieces_c.append(jnp.where(me, prb[:, e:e + 1], 0.0))
            disp_ref[...] = jnp.concatenate(pieces_m, axis=1)
            comb_ref[...] = jnp.concatenate(pieces_c, axis=1)

    return body


def kernel(hidden_states, W):
    b, s, h = hidden_states.shape
    T = b * s
    E = W.shape[0]
    K = 2
    C = max(K, K * T // E)
    NB = 8
    Tb = T // NB
    x = hidden_states.reshape(T, h).astype(jnp.float32)
    wt = W.astype(jnp.float32).T

    comb, disp = pl.pallas_call(
        _gate_kernel(T, E, C, NB, Tb),
        grid=(2 * NB,),
        in_specs=[
            pl.BlockSpec((Tb, h), lambda i: (jnp.minimum(i, NB - 1), 0)),
            pl.BlockSpec((h, E), lambda i: (0, 0)),
        ],
        out_specs=[
            pl.BlockSpec((Tb, E * C), lambda i: (jnp.maximum(i - NB, 0), 0)),
            pl.BlockSpec((Tb, E * C), lambda i: (jnp.maximum(i - NB, 0), 0)),
        ],
        out_shape=[
            jax.ShapeDtypeStruct((T, E * C), jnp.float32),
            jax.ShapeDtypeStruct((T, E * C), jnp.bool_),
        ],
        scratch_shapes=[
            pltpu.VMEM((T, E), jnp.float32),
            pltpu.VMEM((T, E), jnp.int32),
            pltpu.VMEM((T, E), jnp.float32),
        ],
        compiler_params=pltpu.CompilerParams(
            dimension_semantics=("arbitrary",),
        ),
    )(x, wt)
    return comb.reshape(T, E, C), disp.reshape(T, E, C)


# speedup vs baseline: 1.9946x; 1.9946x over previous
import jax
import jax.numpy as jnp
from jax.experimental import pallas as pl
from jax.experimental.pallas import tpu as pltpu


def _gate_kernel(T, E, C, NB, Tb):
    def body(x_ref, wt_ref, comb_ref, disp_ref, logits_sc, p_sc, probs_sc):
        i = pl.program_id(0)

        @pl.when(i < NB)
        def _matmul_phase():
            xb = x_ref[...]
            lg = jax.lax.dot_general(
                xb, wt_ref[...], (((1,), (0,)), ((), ())),
                preferred_element_type=jnp.float32)
            logits_sc[pl.ds(i * Tb, Tb), :] = lg

        @pl.when(i == NB - 1)
        def _gating_phase():
            logits = logits_sc[...]
            mx = jnp.max(logits, axis=1, keepdims=True)
            ex = jnp.exp(logits - mx)
            den = jnp.sum(ex, axis=1, keepdims=True)
            gates = ex / den
            idx = jax.lax.broadcasted_iota(jnp.int32, (T, E), 1)
            m1 = jnp.max(gates, axis=1, keepdims=True)
            t1 = jnp.min(jnp.where(gates == m1, idx, E), axis=1, keepdims=True)
            em1 = idx == t1
            g2 = jnp.where(em1, -1.0, gates)
            m2 = jnp.max(g2, axis=1, keepdims=True)
            t2 = jnp.min(jnp.where(g2 == m2, idx, E), axis=1, keepdims=True)
            em2 = idx == t2
            gs = jnp.maximum(m1 + m2, jnp.finfo(jnp.float32).eps)
            probs_sc[...] = gates / gs
            cnt = jnp.concatenate(
                [em1.astype(jnp.int32), em2.astype(jnp.int32)], axis=1)
            c = cnt
            s = 1
            while s < T:
                c = c + jnp.concatenate(
                    [jnp.zeros((s, 2 * E), jnp.int32), c[:T - s, :]], axis=0)
                s *= 2
            inc1 = c[:, :E]
            inc2 = c[:, E:]
            excl1 = inc1 - cnt[:, :E]
            excl2 = inc2 - cnt[:, E:]
            total1 = inc1[T - 1:T, :]
            p = jnp.where(em1, excl1,
                          jnp.where(em2, total1 + excl2, -1))
            p_sc[...] = p

        @pl.when(i >= NB)
        def _write_phase():
            bb = i - NB
            pb = p_sc[pl.ds(bb * Tb, Tb), :]
            prb = probs_sc[pl.ds(bb * Tb, Tb), :]
            pe = jnp.where(jnp.logical_and(pb >= 0, pb < C), pb, C)
            ci3 = jax.lax.broadcasted_iota(jnp.int32, (Tb, E, C), 2)
            me3 = ci3 == pe[:, :, None]
            disp_ref[...] = me3
            comb_ref[...] = jnp.where(me3, prb[:, :, None], 0.0)

    return body


def kernel(hidden_states, W):
    b, s, h = hidden_states.shape
    T = b * s
    E = W.shape[0]
    K = 2
    C = max(K, K * T // E)
    NB = 8
    Tb = T // NB
    x = hidden_states.reshape(T, h).astype(jnp.float32)
    wt = W.astype(jnp.float32).T

    comb, disp = pl.pallas_call(
        _gate_kernel(T, E, C, NB, Tb),
        grid=(2 * NB,),
        in_specs=[
            pl.BlockSpec((Tb, h), lambda i: (jnp.minimum(i, NB - 1), 0)),
            pl.BlockSpec((h, E), lambda i: (0, 0)),
        ],
        out_specs=[
            pl.BlockSpec((Tb, E, C), lambda i: (jnp.maximum(i - NB, 0), 0, 0)),
            pl.BlockSpec((Tb, E, C), lambda i: (jnp.maximum(i - NB, 0), 0, 0)),
        ],
        out_shape=[
            jax.ShapeDtypeStruct((T, E, C), jnp.float32),
            jax.ShapeDtypeStruct((T, E, C), jnp.bool_),
        ],
        scratch_shapes=[
            pltpu.VMEM((T, E), jnp.float32),
            pltpu.VMEM((T, E), jnp.int32),
            pltpu.VMEM((T, E), jnp.float32),
        ],
        compiler_params=pltpu.CompilerParams(
            dimension_semantics=("arbitrary",),
        ),
    )(x, wt)
    return comb, disp


# speedup vs baseline: 2.5334x; 1.2702x over previous
import jax
import jax.numpy as jnp
from jax.experimental import pallas as pl
from jax.experimental.pallas import tpu as pltpu


def _gate_kernel(T, E, C, NB, Tb):
    def body(x_ref, wt_ref, comb_ref, disp_ref, logits_sc, p_sc, probs_sc):
        i = pl.program_id(0)

        @pl.when(i < NB)
        def _matmul_phase():
            xb = x_ref[...]
            lg = jax.lax.dot_general(
                xb, wt_ref[...], (((1,), (0,)), ((), ())),
                preferred_element_type=jnp.float32)
            logits_sc[pl.ds(i * Tb, Tb), :] = lg

        @pl.when(i == NB - 1)
        def _gating_phase():
            logits = logits_sc[...]
            mx = jnp.max(logits, axis=1, keepdims=True)
            ex = jnp.exp(logits - mx)
            den = jnp.sum(ex, axis=1, keepdims=True)
            gates = ex / den
            idx = jax.lax.broadcasted_iota(jnp.int32, (T, E), 1)
            m1 = jnp.max(gates, axis=1, keepdims=True)
            t1 = jnp.min(jnp.where(gates == m1, idx, E), axis=1, keepdims=True)
            em1 = idx == t1
            g2 = jnp.where(em1, -1.0, gates)
            m2 = jnp.max(g2, axis=1, keepdims=True)
            t2 = jnp.min(jnp.where(g2 == m2, idx, E), axis=1, keepdims=True)
            em2 = idx == t2
            gs = jnp.maximum(m1 + m2, jnp.finfo(jnp.float32).eps)
            probs_sc[...] = gates / gs
            cnt = jnp.concatenate(
                [em1.astype(jnp.int32), em2.astype(jnp.int32)], axis=1)
            c = cnt
            s = 1
            while s < T:
                c = c + jnp.concatenate(
                    [jnp.zeros((s, 2 * E), jnp.int32), c[:T - s, :]], axis=0)
                s *= 2
            inc1 = c[:, :E]
            inc2 = c[:, E:]
            excl1 = inc1 - cnt[:, :E]
            excl2 = inc2 - cnt[:, E:]
            total1 = inc1[T - 1:T, :]
            p = jnp.where(em1, excl1,
                          jnp.where(em2, total1 + excl2, -1))
            p_sc[...] = p

        @pl.when(i >= NB)
        def _write_phase():
            bb = i - NB
            pb = p_sc[pl.ds(bb * Tb, Tb), :]
            prb = probs_sc[pl.ds(bb * Tb, Tb), :]
            pe = jnp.where(jnp.logical_and(pb >= 0, pb < C), pb, C)
            ci3 = jax.lax.broadcasted_iota(jnp.int32, (Tb, E, C), 2)
            me3 = ci3 == pe[:, :, None]
            disp_ref[...] = me3.astype(jnp.int8)
            comb_ref[...] = jnp.where(me3, prb[:, :, None], 0.0)

    return body


def kernel(hidden_states, W):
    b, s, h = hidden_states.shape
    T = b * s
    E = W.shape[0]
    K = 2
    C = max(K, K * T // E)
    NB = 8
    Tb = T // NB
    x = hidden_states.reshape(T, h).astype(jnp.float32)
    wt = W.astype(jnp.float32).T

    comb, disp = pl.pallas_call(
        _gate_kernel(T, E, C, NB, Tb),
        grid=(2 * NB,),
        in_specs=[
            pl.BlockSpec((Tb, h), lambda i: (jnp.minimum(i, NB - 1), 0)),
            pl.BlockSpec((h, E), lambda i: (0, 0)),
        ],
        out_specs=[
            pl.BlockSpec((Tb, E, C), lambda i: (jnp.maximum(i - NB, 0), 0, 0)),
            pl.BlockSpec((Tb, E, C), lambda i: (jnp.maximum(i - NB, 0), 0, 0)),
        ],
        out_shape=[
            jax.ShapeDtypeStruct((T, E, C), jnp.float32),
            jax.ShapeDtypeStruct((T, E, C), jnp.int8),
        ],
        scratch_shapes=[
            pltpu.VMEM((T, E), jnp.float32),
            pltpu.VMEM((T, E), jnp.int32),
            pltpu.VMEM((T, E), jnp.float32),
        ],
        compiler_params=pltpu.CompilerParams(
            dimension_semantics=("arbitrary",),
        ),
    )(x, wt)
    return comb, disp.astype(jnp.bool_)
